# baseline (speedup 1.0000x reference)
import jax
import jax.numpy as jnp
from jax import lax
from jax.experimental import pallas as pl
from jax.experimental.pallas import tpu as pltpu
from jax.experimental.pallas import tpu_sc as plsc

NC = 2
NS = 16
NW = NC * NS
L = 16
C = 128


def _bcast(vec, j):
    idx = jnp.full((L, 1), j, jnp.int32)
    dn = lax.GatherDimensionNumbers(offset_dims=(), collapsed_slice_dims=(0,),
                                    start_index_map=(0,))
    return lax.gather(vec, idx, dn, slice_sizes=(1,),
                      mode=lax.GatherScatterMode.PROMISE_IN_BOUNDS)


def _pla_body(u_hbm, i_hbm, rst_hbm, p_hbm, q_hbm, th_hbm, bias_hbm,
              rhat_hbm, at_hbm,
              idxu_v, idxi_v, pu_bufs, qi_bufs, th_v, rs_v, acc_v,
              rhat_v, av_v, bias_v, sem_g, sem):
    num_models, two_k = th_hbm.shape
    k_dim = two_k // 2
    b = u_hbm.shape[0]
    b_per_w = b // NW
    n_chunks = b_per_w // C
    n_g = C // L

    wid = lax.axis_index("s") * NC + lax.axis_index("c")
    wbase = wid * b_per_w

    pltpu.sync_copy(th_hbm, th_v)
    pltpu.sync_copy(bias_hbm, bias_v.at[pl.ds(0, 1)])
    pltpu.sync_copy(u_hbm.at[pl.ds(wbase, b_per_w)], idxu_v)
    pltpu.sync_copy(i_hbm.at[pl.ds(wbase, b_per_w)], idxi_v)
    bias = bias_v[pl.ds(0, L)][0]
    lanes = lax.iota(jnp.int32, L)

    def fire(c):
        s = c % 2
        gp = pltpu.async_copy(p_hbm.at[idxu_v.at[pl.ds(c * C, C)]],
                              pu_bufs[s], sem_g)
        gq = pltpu.async_copy(q_hbm.at[idxi_v.at[pl.ds(c * C, C)]],
                              qi_bufs[s], sem_g)
        return gp, gq

    pend = fire(0)
    for c in range(n_chunks):
        base = wbase + c * C
        pend[0].wait()
        pend[1].wait()
        if c + 1 < n_chunks:
            pend = fire(c + 1)
        pu_v, qi_v = pu_bufs[c % 2], qi_bufs[c % 2]
        pltpu.sync_copy(rst_hbm.at[:, pl.ds(base, C)], rs_v)

        def kc_step(kc, _):
            thu_vec = [th_v[m, pl.ds(kc * L, L)] for m in range(num_models)]
            thi_vec = [th_v[m, pl.ds(k_dim + kc * L, L)]
                       for m in range(num_models)]

            def j_step(j, t):
                col = jnp.full((L,), kc * L + j, jnp.int32)
                bu = [_bcast(thu_vec[m], j) for m in range(num_models)]
                bi = [_bcast(thi_vec[m], j) for m in range(num_models)]
                t = list(t)
                for g in range(n_g):
                    rows = lanes + g * L
                    pu_k = plsc.load_gather(pu_v, [rows, col])
                    qi_k = plsc.load_gather(qi_v, [rows, col])
                    for m in range(num_models):
                        t[g * num_models + m] = (t[g * num_models + m]
                                                 + pu_k * bu[m]
                                                 + qi_k * bi[m])
                return tuple(t)

            zeros = tuple(jnp.zeros((L,), jnp.float32)
                          for _ in range(n_g * num_models))
            t = lax.fori_loop(0, L, j_step, zeros)
            for g in range(n_g):
                for m in range(num_models):
                    plsc.addupdate(acc_v.at[m, pl.ds(g * L, L)],
                                   t[g * num_models + m])
            return 0

        zero = jnp.zeros((L,), jnp.float32)
        for m in range(num_models):
            for g in range(n_g):
                acc_v[m, pl.ds(g * L, L)] = zero
        lax.fori_loop(0, k_dim // L, kc_step, 0)

        def out_step(g, _):
            sl = pl.ds(g * L, L)
            logits = [acc_v[m, sl] for m in range(num_models)]
            mx = logits[0]
            for m in range(1, num_models):
                mx = jnp.maximum(mx, logits[m])
            es = [jnp.exp(lg - mx) for lg in logits]
            tot = es[0]
            for m in range(1, num_models):
                tot = tot + es[m]
            r = jnp.zeros((L,), jnp.float32)
            for m in range(num_models):
                a_m = es[m] / tot
                av_v[m, sl] = a_m
                r = r + a_m * rs_v[m, sl]
            rhat_v[sl] = r + bias
            return 0

        lax.fori_loop(0, n_g, out_step, 0)
        pltpu.sync_copy(rhat_v, rhat_hbm.at[pl.ds(base, C)])
        pltpu.sync_copy(av_v, at_hbm.at[:, pl.ds(base, C)])


@jax.jit
def _pla_sc(u32, i32, rst, p, q, theta, bias):
    b = u32.shape[0]
    num_models, two_k = theta.shape
    k_dim = two_k // 2
    mesh = plsc.VectorSubcoreMesh(core_axis_name="c", subcore_axis_name="s",
                                  num_cores=NC, num_subcores=NS)
    return pl.kernel(
        _pla_body,
        out_type=[
            jax.ShapeDtypeStruct((b,), jnp.float32),
            jax.ShapeDtypeStruct((num_models, b), jnp.float32),
        ],
        mesh=mesh,
        compiler_params=pltpu.CompilerParams(needs_layout_passes=False),
        scratch_types=[
            pltpu.VMEM((b // NW,), jnp.int32),
            pltpu.VMEM((b // NW,), jnp.int32),
            [pltpu.VMEM((C, k_dim), jnp.float32) for _ in range(2)],
            [pltpu.VMEM((C, k_dim), jnp.float32) for _ in range(2)],
            pltpu.VMEM((num_models, two_k), jnp.float32),
            pltpu.VMEM((num_models, C), jnp.float32),
            pltpu.VMEM((num_models, C), jnp.float32),
            pltpu.VMEM((C,), jnp.float32),
            pltpu.VMEM((num_models, C), jnp.float32),
            pltpu.VMEM((L,), jnp.float32),
            pltpu.SemaphoreType.DMA,
            pltpu.SemaphoreType.DMA,
        ],
    )(u32, i32, rst, p, q, theta, bias)


def kernel(u_idx, i_idx, r_s, P, Q, theta, bias):
    u32 = u_idx.astype(jnp.int32)
    i32 = i_idx.astype(jnp.int32)
    r_hat, alphas_t = _pla_sc(u32, i32, r_s.T, P, Q, theta, bias)
    return (r_hat, alphas_t.T, r_s)


# speedup vs baseline: 3.0273x; 3.0273x over previous
import functools

import jax
import jax.numpy as jnp
from jax import lax
from jax.experimental import pallas as pl
from jax.experimental.pallas import tpu as pltpu
from jax.experimental.pallas import tpu_sc as plsc

NC = 2
NS = 16
NW = NC * NS

C = 128
RB = 2048


def _gather_body(u_hbm, i_hbm, p_hbm, q_hbm, pu_hbm, qi_hbm,
                 idxu_v, idxi_v, bufs, sem_g, sem_w):
    b = u_hbm.shape[0]
    b_per_w = b // NW
    n_chunks = b_per_w // C
    n_t = 2 * n_chunks
    n_buf = len(bufs)

    wid = lax.axis_index("s") * NC + lax.axis_index("c")
    wbase = wid * b_per_w

    pltpu.sync_copy(u_hbm.at[pl.ds(wbase, b_per_w)], idxu_v)
    pltpu.sync_copy(i_hbm.at[pl.ds(wbase, b_per_w)], idxi_v)

    def plan(t):
        if t < n_chunks:
            return p_hbm, idxu_v, pu_hbm, t
        return q_hbm, idxi_v, qi_hbm, t - n_chunks

    g_h = [None] * n_t
    w_h = [None] * n_t
    for t in range(n_t):
        if t >= n_buf:
            w_h[t - n_buf].wait()
        tab, idxv, _, c = plan(t)
        g_h[t] = pltpu.async_copy(tab.at[idxv.at[pl.ds(c * C, C)]],
                                  bufs[t % n_buf], sem_g)
        if t >= 1:
            g_h[t - 1].wait()
            _, _, out, cp = plan(t - 1)
            w_h[t - 1] = pltpu.async_copy(
                bufs[(t - 1) % n_buf], out.at[pl.ds(wbase + cp * C, C)],
                sem_w)
    g_h[n_t - 1].wait()
    _, _, out, cp = plan(n_t - 1)
    w_h[n_t - 1] = pltpu.async_copy(
        bufs[(n_t - 1) % n_buf], out.at[pl.ds(wbase + cp * C, C)], sem_w)
    for t in range(n_t - n_buf, n_t):
        w_h[t].wait()


def _gather_sc(u32, i32, p, q):
    b = u32.shape[0]
    k_dim = p.shape[1]
    mesh = plsc.VectorSubcoreMesh(core_axis_name="c", subcore_axis_name="s",
                                  num_cores=NC, num_subcores=NS)
    return pl.kernel(
        _gather_body,
        out_type=[
            jax.ShapeDtypeStruct((b, k_dim), jnp.float32),
            jax.ShapeDtypeStruct((b, k_dim), jnp.float32),
        ],
        mesh=mesh,
        compiler_params=pltpu.CompilerParams(needs_layout_passes=False),
        scratch_types=[
            pltpu.VMEM((b // NW,), jnp.int32),
            pltpu.VMEM((b // NW,), jnp.int32),
            [pltpu.VMEM((C, k_dim), jnp.float32) for _ in range(4)],
            pltpu.SemaphoreType.DMA,
            pltpu.SemaphoreType.DMA,
        ],
    )(u32, i32, p, q)


def _dense_body(pu_ref, qi_ref, rst_ref, th_ref, bias_ref, rhat_ref, alt_ref):
    k_dim = pu_ref.shape[1]
    th = th_ref[...]
    dn = (((1,), (1,)), ((), ()))
    lt = lax.dot_general(th[:, :k_dim], pu_ref[...], dn,
                         preferred_element_type=jnp.float32)
    lt += lax.dot_general(th[:, k_dim:], qi_ref[...], dn,
                          preferred_element_type=jnp.float32)
    mx = jnp.max(lt, axis=0, keepdims=True)
    e = jnp.exp(lt - mx)
    al = e / jnp.sum(e, axis=0, keepdims=True)
    alt_ref[...] = al
    rhat_ref[...] = (jnp.sum(al * rst_ref[...], axis=0, keepdims=True)
                     + bias_ref[0])


def _dense_tc(pu, qi, rst, theta, bias):
    b, k_dim = pu.shape
    num_models = rst.shape[0]
    grid = (b // RB,)
    return pl.pallas_call(
        _dense_body,
        grid=grid,
        in_specs=[
            pl.BlockSpec((RB, k_dim), lambda i: (i, 0)),
            pl.BlockSpec((RB, k_dim), lambda i: (i, 0)),
            pl.BlockSpec((num_models, RB), lambda i: (0, i)),
            pl.BlockSpec((num_models, 2 * k_dim), lambda i: (0, 0)),
            pl.BlockSpec(memory_space=pltpu.SMEM),
        ],
        out_specs=[
            pl.BlockSpec((1, RB), lambda i: (0, i)),
            pl.BlockSpec((num_models, RB), lambda i: (0, i)),
        ],
        out_shape=[
            jax.ShapeDtypeStruct((1, b), jnp.float32),
            jax.ShapeDtypeStruct((num_models, b), jnp.float32),
        ],
    )(pu, qi, rst, theta, bias)


@jax.jit
def _pla(u_idx, i_idx, r_s, p, q, theta, bias):
    u32 = u_idx.astype(jnp.int32)
    i32 = i_idx.astype(jnp.int32)
    pu, qi = _gather_sc(u32, i32, p, q)
    rhat2, alt = _dense_tc(pu, qi, r_s.T, theta, bias)
    return rhat2.reshape(r_s.shape[0]), alt.T


def kernel(u_idx, i_idx, r_s, P, Q, theta, bias):
    r_hat, alphas = _pla(u_idx, i_idx, r_s, P, Q, theta, bias)
    return (r_hat, alphas, r_s)
